# baseline (speedup 1.0000x reference)
import functools

import jax
import jax.numpy as jnp
from jax import lax
from jax.experimental import pallas as pl
from jax.experimental.pallas import tpu as pltpu
from jax.experimental.pallas import tpu_sc as plsc

N = 10000
E = 320000
D = 128
H = 128
C = 64

NC = 2
NS = 16
NW = NC * NS
EPT = E // NW
KC = 125
NCHUNK = EPT // KC
NBUF = 8
RPT = 632
RPT_LAST = N - 15 * RPT

_mesh = plsc.VectorSubcoreMesh(core_axis_name="c", subcore_axis_name="s")


@functools.partial(
    pl.kernel,
    mesh=_mesh,
    compiler_params=pltpu.CompilerParams(use_tc_tiling_on_sc=False),
    out_type=[jax.ShapeDtypeStruct((N, D), jnp.bfloat16)] * 2,
    scratch_types=[
        pltpu.VMEM((2, NCHUNK, KC), jnp.int32),
        pltpu.VMEM((NBUF, KC, D), jnp.bfloat16),
        pltpu.VMEM_SHARED((N, D), jnp.bfloat16),
        pltpu.SemaphoreType.DMA,
        pltpu.SemaphoreType.DMA((NBUF,)),
    ],
)
def _segsum_sc(x_hbm, src_hbm, dst_hbm, zeros_hbm, o0, o1,
               idx_all, rows, acc, semg, sems):
    outs = (o0, o1)
    c = lax.axis_index("c")
    s = lax.axis_index("s")
    wid = s * NC + c
    r0 = pl.multiple_of(s * RPT, 8)

    pltpu.sync_copy(src_hbm.at[wid], idx_all.at[0])
    pltpu.sync_copy(dst_hbm.at[wid], idx_all.at[1])

    @pl.when(s < NS - 1)
    def _():
        pltpu.sync_copy(zeros_hbm, acc.at[pl.ds(r0, RPT)])

    @pl.when(s == NS - 1)
    def _():
        pltpu.sync_copy(zeros_hbm.at[pl.ds(0, RPT_LAST)],
                        acc.at[pl.ds((NS - 1) * RPT, RPT_LAST)])

    for b in range(NBUF):
        pltpu.async_copy(x_hbm.at[idx_all.at[0, b]], rows.at[b], semg)

    plsc.subcore_barrier()

    def outer(t, carry):
        for b in range(NBUF):
            i = t * NBUF + b
            pltpu.make_async_copy(x_hbm.at[idx_all.at[0, i]],
                                  rows.at[b], semg).wait()
            pltpu.make_async_copy(rows.at[b],
                                  acc.at[idx_all.at[1, i]],
                                  sems.at[b]).start(add=True)
            bp = (b - 2) % NBUF

            @pl.when(i >= 2)
            def _():
                pltpu.make_async_copy(rows.at[bp],
                                      acc.at[idx_all.at[1, i]],
                                      sems.at[bp]).wait()
                j = jnp.minimum(i - 2 + NBUF, NCHUNK - 1)
                pltpu.async_copy(x_hbm.at[idx_all.at[0, j]],
                                 rows.at[bp], semg)
        return carry

    lax.fori_loop(0, NCHUNK // NBUF, outer, 0)
    for k in (NCHUNK - 2, NCHUNK - 1):
        pltpu.make_async_copy(rows.at[k % NBUF],
                              acc.at[idx_all.at[1, k]],
                              sems.at[k % NBUF]).wait()
    for b in range(NBUF - 2):
        pltpu.make_async_copy(x_hbm.at[idx_all.at[0, 0]], rows.at[b],
                              semg).wait()
    plsc.subcore_barrier()

    for cc in range(NC):
        @pl.when((c == cc) & (s < NS - 1))
        def _():
            pltpu.sync_copy(acc.at[pl.ds(r0, RPT)],
                            outs[cc].at[pl.ds(r0, RPT)])

        @pl.when((c == cc) & (s == NS - 1))
        def _():
            pltpu.sync_copy(acc.at[pl.ds((NS - 1) * RPT, RPT_LAST)],
                            outs[cc].at[pl.ds((NS - 1) * RPT, RPT_LAST)])


def _mlp1(x, p0, p1, W1, b1):
    BM = 1000

    def body(x_ref, a0, a1, w_ref, bias, h_ref):
        f32 = jnp.float32
        sm = x_ref[...] + a0[...].astype(f32) + a1[...].astype(f32)
        z = jnp.dot(sm, w_ref[...], preferred_element_type=jnp.float32)
        h_ref[...] = jnp.maximum(z + bias[...], 0.0).astype(jnp.bfloat16)

    blk = pl.BlockSpec((BM, D), lambda i: (i, 0))
    return pl.pallas_call(
        body,
        grid=(N // BM,),
        in_specs=[blk, blk, blk,
                  pl.BlockSpec((D, H), lambda i: (0, 0)),
                  pl.BlockSpec((1, H), lambda i: (0, 0))],
        out_specs=pl.BlockSpec((BM, H), lambda i: (i, 0)),
        out_shape=jax.ShapeDtypeStruct((N, H), jnp.bfloat16),
    )(x, p0, p1, W1, b1.reshape(1, H))


def _mlp2(h, q0, q1, W2, b2):
    BM = 1000

    def body(h_ref, a0, a1, w_ref, bias, o_ref):
        f32 = jnp.float32
        sm = (h_ref[...].astype(f32) + a0[...].astype(f32)
              + a1[...].astype(f32))
        z = jnp.dot(sm, w_ref[...], preferred_element_type=jnp.float32)
        z = z + bias[...]
        m = jnp.max(z, axis=-1, keepdims=True)
        e = z - m
        lse = jnp.log(jnp.sum(jnp.exp(e), axis=-1, keepdims=True))
        o_ref[...] = e - lse

    blk = pl.BlockSpec((BM, H), lambda i: (i, 0))
    return pl.pallas_call(
        body,
        grid=(N // BM,),
        in_specs=[blk, blk, blk,
                  pl.BlockSpec((H, C), lambda i: (0, 0)),
                  pl.BlockSpec((1, C), lambda i: (0, 0))],
        out_specs=pl.BlockSpec((BM, C), lambda i: (i, 0)),
        out_shape=jax.ShapeDtypeStruct((N, C), jnp.float32),
    )(h, q0, q1, W2, b2.reshape(1, C))


def kernel(x, edge_index, W1, b1, W2, b2):
    src = edge_index[0].astype(jnp.int32).reshape(NW, NCHUNK, KC)
    dst = edge_index[1].astype(jnp.int32).reshape(NW, NCHUNK, KC)
    zeros = jnp.zeros((RPT, D), jnp.bfloat16)
    xb = x.astype(jnp.bfloat16)
    p0, p1 = _segsum_sc(xb, src, dst, zeros)
    h = _mlp1(x, p0, p1, W1, b1)
    q0, q1 = _segsum_sc(h, src, dst, zeros)
    return _mlp2(h, q0, q1, W2, b2)


# speedup vs baseline: 1.0779x; 1.0779x over previous
import functools

import jax
import jax.numpy as jnp
from jax import lax
from jax.experimental import pallas as pl
from jax.experimental.pallas import tpu as pltpu
from jax.experimental.pallas import tpu_sc as plsc

N = 10000
E = 320000
D = 128
H = 128
C = 64

NC = 2
NS = 16
NW = NC * NS
EPT = E // NW
KC = 125
NCHUNK = EPT // KC
NBUF = 4
RPT = 632
RPT_LAST = N - 15 * RPT

_mesh = plsc.VectorSubcoreMesh(core_axis_name="c", subcore_axis_name="s")


@functools.partial(
    pl.kernel,
    mesh=_mesh,
    compiler_params=pltpu.CompilerParams(use_tc_tiling_on_sc=False),
    out_type=[jax.ShapeDtypeStruct((N, D), jnp.bfloat16)] * 2,
    scratch_types=[
        pltpu.VMEM((2, NCHUNK, KC), jnp.int32),
        pltpu.VMEM((NBUF, KC, D), jnp.bfloat16),
        pltpu.VMEM_SHARED((N, D), jnp.bfloat16),
        pltpu.SemaphoreType.DMA,
        pltpu.SemaphoreType.DMA((NBUF,)),
    ],
)
def _segsum_sc(x_hbm, eidx_hbm, zeros_hbm, o0, o1,
               idx_all, rows, acc, semg, sems):
    outs = (o0, o1)
    c = lax.axis_index("c")
    s = lax.axis_index("s")
    wid = s * NC + c
    r0 = pl.multiple_of(s * RPT, 8)

    pltpu.sync_copy(eidx_hbm.at[0, wid], idx_all.at[0])
    pltpu.sync_copy(eidx_hbm.at[1, wid], idx_all.at[1])

    @pl.when(s < NS - 1)
    def _():
        pltpu.sync_copy(zeros_hbm, acc.at[pl.ds(r0, RPT)])

    @pl.when(s == NS - 1)
    def _():
        pltpu.sync_copy(zeros_hbm.at[pl.ds(0, RPT_LAST)],
                        acc.at[pl.ds((NS - 1) * RPT, RPT_LAST)])

    for b in range(NBUF):
        pltpu.async_copy(x_hbm.at[idx_all.at[0, b]], rows.at[b], semg)

    plsc.subcore_barrier()

    def outer(t, carry):
        for b in range(NBUF):
            i = t * NBUF + b
            pltpu.make_async_copy(x_hbm.at[idx_all.at[0, i]],
                                  rows.at[b], semg).wait()
            pltpu.make_async_copy(rows.at[b],
                                  acc.at[idx_all.at[1, i]],
                                  sems.at[b]).start(add=True)
            bp = (b - 1) % NBUF

            @pl.when(i >= 1)
            def _():
                pltpu.make_async_copy(rows.at[bp],
                                      acc.at[idx_all.at[1, i]],
                                      sems.at[bp]).wait()
                j = jnp.minimum(i - 1 + NBUF, NCHUNK - 1)
                pltpu.async_copy(x_hbm.at[idx_all.at[0, j]],
                                 rows.at[bp], semg)
        return carry

    lax.fori_loop(0, NCHUNK // NBUF, outer, 0)
    pltpu.make_async_copy(rows.at[(NCHUNK - 1) % NBUF],
                          acc.at[idx_all.at[1, NCHUNK - 1]],
                          sems.at[(NCHUNK - 1) % NBUF]).wait()
    for b in range(NBUF - 1):
        pltpu.make_async_copy(x_hbm.at[idx_all.at[0, 0]], rows.at[b],
                              semg).wait()
    plsc.subcore_barrier()

    for cc in range(NC):
        @pl.when((c == cc) & (s < NS - 1))
        def _():
            pltpu.sync_copy(acc.at[pl.ds(r0, RPT)],
                            outs[cc].at[pl.ds(r0, RPT)])

        @pl.when((c == cc) & (s == NS - 1))
        def _():
            pltpu.sync_copy(acc.at[pl.ds((NS - 1) * RPT, RPT_LAST)],
                            outs[cc].at[pl.ds((NS - 1) * RPT, RPT_LAST)])


def _mlp1(x, p0, p1, W1, b1):
    BM = 1000

    def body(x_ref, a0, a1, w_ref, bias, h_ref):
        f32 = jnp.float32
        sm = x_ref[...] + a0[...].astype(f32) + a1[...].astype(f32)
        z = jnp.dot(sm, w_ref[...], preferred_element_type=jnp.float32)
        h_ref[...] = jnp.maximum(z + bias[...], 0.0).astype(jnp.bfloat16)

    blk = pl.BlockSpec((BM, D), lambda i: (i, 0))
    return pl.pallas_call(
        body,
        grid=(N // BM,),
        in_specs=[blk, blk, blk,
                  pl.BlockSpec((D, H), lambda i: (0, 0)),
                  pl.BlockSpec((1, H), lambda i: (0, 0))],
        out_specs=pl.BlockSpec((BM, H), lambda i: (i, 0)),
        out_shape=jax.ShapeDtypeStruct((N, H), jnp.bfloat16),
    )(x, p0, p1, W1, b1.reshape(1, H))


def _mlp2(h, q0, q1, W2, b2):
    BM = 1000

    def body(h_ref, a0, a1, w_ref, bias, o_ref):
        f32 = jnp.float32
        sm = (h_ref[...].astype(f32) + a0[...].astype(f32)
              + a1[...].astype(f32))
        z = jnp.dot(sm, w_ref[...], preferred_element_type=jnp.float32)
        z = z + bias[...]
        m = jnp.max(z, axis=-1, keepdims=True)
        e = z - m
        lse = jnp.log(jnp.sum(jnp.exp(e), axis=-1, keepdims=True))
        o_ref[...] = e - lse

    blk = pl.BlockSpec((BM, H), lambda i: (i, 0))
    return pl.pallas_call(
        body,
        grid=(N // BM,),
        in_specs=[blk, blk, blk,
                  pl.BlockSpec((H, C), lambda i: (0, 0)),
                  pl.BlockSpec((1, C), lambda i: (0, 0))],
        out_specs=pl.BlockSpec((BM, C), lambda i: (i, 0)),
        out_shape=jax.ShapeDtypeStruct((N, C), jnp.float32),
    )(h, q0, q1, W2, b2.reshape(1, C))


def kernel(x, edge_index, W1, b1, W2, b2):
    eidx = edge_index.astype(jnp.int32).reshape(2, NW, NCHUNK, KC)
    zeros = jnp.zeros((RPT, D), jnp.bfloat16)
    xb = x.astype(jnp.bfloat16)
    p0, p1 = _segsum_sc(xb, eidx, zeros)
    h = _mlp1(x, p0, p1, W1, b1)
    q0, q1 = _segsum_sc(h, eidx, zeros)
    return _mlp2(h, q0, q1, W2, b2)
